# baseline (speedup 1.0000x reference)
import functools

import jax
import jax.numpy as jnp
import numpy as np
from jax import lax
from jax.experimental import pallas as pl
from jax.experimental.pallas import tpu as pltpu
from jax.experimental.pallas import tpu_sc as plsc

_N_ENT = 10000
_N_REL = 16
_DIM = 128
_HD = _DIM // 2
_N_EDGES = 320000

_NC, _NS = 2, 16
_NP = 10112
_NB = _NP // 128
_NKEY = _N_ENT * _N_REL
_NT = (_N_REL + 1) * _NP

_K = 80
_EPT = _N_EDGES // _NS
_NCHUNK = _EPT // _K
_NRING = 5
_ECHUNK = _N_EDGES // _K

_ROWS_PER_TILE = _NP // _NS

_PW = np.arange(_DIM)
_PORIG = 32 * (_PW // 32) + (_PW % 32) // 2 + 16 * (_PW % 2)
_PERM = np.zeros((_DIM, _DIM), dtype=np.float32)
_PERM[_PORIG, _PW] = 1.0


def _wstack_body(comp_ref, basis_ref, p_ref, o_ref):
    t = jnp.dot(comp_ref[...], basis_ref[...],
                preferred_element_type=jnp.float32)
    tr = t.reshape(_N_REL, _DIM, _DIM)
    p = p_ref[...]
    for r in range(_N_REL):
        o_ref[r] = jnp.dot(tr[r], p, preferred_element_type=jnp.float32)


def _wstack(comp, basis, perm):
    return pl.pallas_call(
        _wstack_body,
        out_shape=jax.ShapeDtypeStruct((_N_REL, _DIM, _DIM), jnp.float32),
    )(comp, basis.reshape(4, _DIM * _DIM), perm)


_BR = _NP // 8


def _prep_body(ei_ref, et_ref, okey_ref, ock_ref, odst_ref):
    src = ei_ref[0]
    dst = ei_ref[1]
    rel = et_ref[...]
    key = (rel * _NP + src) * 2
    okey_ref[0] = key
    okey_ref[1] = key + 1
    ock_ref[...] = dst * _N_REL + rel
    odst_ref[...] = dst


def _prep_edges(edge_index, edge_type):
    er = _N_EDGES // 128
    return pl.pallas_call(
        _prep_body,
        out_shape=[
            jax.ShapeDtypeStruct((_NC, er, 128), jnp.int32),
            jax.ShapeDtypeStruct((er, 128), jnp.int32),
            jax.ShapeDtypeStruct((er, 128), jnp.int32),
        ],
    )(edge_index.reshape(2, er, 128), edge_type.reshape(er, 128))


def _relmm_body(x_ref, w_ref, obf_ref, oroot_ref, *, off):
    x = x_ref[...]
    for r in range(_N_REL):
        y = jnp.dot(x, w_ref[off + r], preferred_element_type=jnp.float32)
        obf_ref[r] = y.astype(jnp.bfloat16)
    oroot_ref[...] = jnp.dot(x, w_ref[off + _N_REL],
                             preferred_element_type=jnp.float32)


def _rel_matmul(x_pad, wstack, off):
    return pl.pallas_call(
        functools.partial(_relmm_body, off=off),
        grid=(8,),
        in_specs=[
            pl.BlockSpec((_BR, _DIM), lambda i: (i, 0)),
            pl.BlockSpec((2 * _N_REL + 2, _DIM, _DIM), lambda i: (0, 0, 0)),
        ],
        out_specs=[
            pl.BlockSpec((_N_REL, _BR, _DIM), lambda i: (0, i, 0)),
            pl.BlockSpec((_BR, _DIM), lambda i: (i, 0)),
        ],
        out_shape=[
            jax.ShapeDtypeStruct((_N_REL, _NP, _DIM), jnp.bfloat16),
            jax.ShapeDtypeStruct((_NP, _DIM), jnp.float32),
        ],
    )(x_pad, wstack)


def _relmm_fused_body(p_ref, r_ref, b_ref, w_ref, obf_ref, oroot_ref, *, off):
    x = jnp.maximum(p_ref[...] + r_ref[...] + b_ref[...], 0.0)
    for r in range(_N_REL):
        y = jnp.dot(x, w_ref[off + r], preferred_element_type=jnp.float32)
        obf_ref[r] = y.astype(jnp.bfloat16)
    oroot_ref[...] = jnp.dot(x, w_ref[off + _N_REL],
                             preferred_element_type=jnp.float32)


def _rel_matmul_fused(p_flat, root_prev, bias, wstack, off):
    return pl.pallas_call(
        functools.partial(_relmm_fused_body, off=off),
        grid=(8,),
        in_specs=[
            pl.BlockSpec((_BR, _DIM), lambda i: (i, 0)),
            pl.BlockSpec((_BR, _DIM), lambda i: (i, 0)),
            pl.BlockSpec((1, _DIM), lambda i: (0, 0)),
            pl.BlockSpec((2 * _N_REL + 2, _DIM, _DIM), lambda i: (0, 0, 0)),
        ],
        out_specs=[
            pl.BlockSpec((_N_REL, _BR, _DIM), lambda i: (0, i, 0)),
            pl.BlockSpec((_BR, _DIM), lambda i: (i, 0)),
        ],
        out_shape=[
            jax.ShapeDtypeStruct((_N_REL, _NP, _DIM), jnp.bfloat16),
            jax.ShapeDtypeStruct((_NP, _DIM), jnp.float32),
        ],
    )(p_flat, root_prev, bias.reshape(1, _DIM), wstack)


def _combine_body(p_ref, r_ref, b_ref, o_ref):
    o_ref[...] = p_ref[...] + r_ref[...] + b_ref[...]


def _combine(p_flat, root, bias):
    return pl.pallas_call(
        _combine_body,
        grid=(8,),
        in_specs=[
            pl.BlockSpec((_BR, _DIM), lambda i: (i, 0)),
            pl.BlockSpec((_BR, _DIM), lambda i: (i, 0)),
            pl.BlockSpec((1, _DIM), lambda i: (0, 0)),
        ],
        out_specs=pl.BlockSpec((_BR, _DIM), lambda i: (i, 0)),
        out_shape=jax.ShapeDtypeStruct((_NP, _DIM), jnp.float32),
    )(p_flat, root, bias.reshape(1, _DIM))


def _norm_body(ckey_hbm, inv_hbm, counts_sh, zbuf, ckeys_v, cnts_v, ones_v,
               sem_s, sem_g):
    c = lax.axis_index("c")
    s = lax.axis_index("s")

    def zb(i, _):
        zbuf[pl.ds(i * 16, 16)] = jnp.zeros((16,), jnp.float32)
        return 0
    lax.fori_loop(0, zbuf.shape[0] // 16, zb, 0)

    def ob(i, _):
        ones_v[pl.ds(i * 16, 16)] = jnp.ones((16,), jnp.float32)
        return 0
    lax.fori_loop(0, _K // 16, ob, 0)

    pltpu.sync_copy(zbuf, counts_sh.at[pl.ds(s * (_NKEY // _NS),
                                             _NKEY // _NS)])
    plsc.subcore_barrier()

    pltpu.sync_copy(ckey_hbm.at[pl.ds(s * _NCHUNK, _NCHUNK)], ckeys_v)

    def f1(cc, _):
        pltpu.async_copy(ones_v, counts_sh.at[ckeys_v.at[cc]], sem_s,
                         add=True)
        return 0
    lax.fori_loop(0, _NCHUNK, f1, 0)

    def d1(cc, _):
        pltpu.make_async_copy(ones_v, counts_sh.at[ckeys_v.at[0]],
                              sem_s).wait()
        return 0
    lax.fori_loop(0, _NCHUNK, d1, 0)
    plsc.subcore_barrier()

    nch2 = _NCHUNK // 2
    row0 = (c * _NS + s) * nch2
    pltpu.sync_copy(ckey_hbm.at[pl.ds(row0, nch2)],
                    ckeys_v.at[pl.ds(0, nch2)])

    def f2(cc, _):
        pltpu.async_copy(counts_sh.at[ckeys_v.at[cc]], cnts_v.at[cc], sem_g)
        return 0
    lax.fori_loop(0, nch2, f2, 0)

    def d2(cc, _):
        pltpu.make_async_copy(counts_sh.at[ckeys_v.at[0]], cnts_v.at[0],
                              sem_g).wait()
        return 0
    lax.fori_loop(0, nch2, d2, 0)

    def inv(i, _):
        for g in range(_K // 16):
            sl = pl.ds(g * 16, 16)
            cnts_v[i, sl] = 1.0 / jnp.maximum(cnts_v[i, sl], 1.0)
        return 0
    lax.fori_loop(0, nch2, inv, 0)

    pltpu.sync_copy(cnts_v, inv_hbm.at[pl.ds(row0, nch2)])


def _edge_norms(ckey2):
    mesh = plsc.VectorSubcoreMesh(core_axis_name="c", subcore_axis_name="s")
    f = pl.kernel(
        _norm_body,
        out_type=jax.ShapeDtypeStruct((_ECHUNK, _K), jnp.float32),
        mesh=mesh,
        scratch_types=[
            pltpu.VMEM_SHARED((_NKEY,), jnp.float32),
            pltpu.VMEM((_NKEY // _NS,), jnp.float32),
            pltpu.VMEM((_NCHUNK, _K), jnp.int32),
            pltpu.VMEM((_NCHUNK // 2, _K), jnp.float32),
            pltpu.VMEM((_K,), jnp.float32),
            pltpu.SemaphoreType.DMA,
            pltpu.SemaphoreType.DMA,
        ],
        compiler_params=pltpu.CompilerParams(use_tc_tiling_on_sc=False),
    )
    return f(ckey2)


_NTOT = 252


def _agg_body(h_hbm, key_hbm, dst_hbm, inv_hbm, out_hbm, agg_sh,
              rin, rout, keys, invs, dsts, zbuf,
              sem_z, sem_g, sem_s, sem_i, sem_d):
    c = lax.axis_index("c")
    s = lax.axis_index("s")
    row_base = s * _NCHUNK

    def crow(cc):
        return row_base + jnp.minimum(cc, _NCHUNK - 1)

    def fire_ki(cc, q):
        r = crow(cc)
        pltpu.async_copy(key_hbm.at[c, r], keys[q], sem_i[q])
        pltpu.async_copy(inv_hbm.at[r], invs[q], sem_i[q])

    def wait_ki(q):
        pltpu.make_async_copy(key_hbm.at[c, 0], keys[q], sem_i[q]).wait()
        pltpu.make_async_copy(inv_hbm.at[0], invs[q], sem_i[q]).wait()

    def fire_d(cc, q):
        pltpu.async_copy(dst_hbm.at[crow(cc)], dsts[q], sem_d[q])

    def wait_d(q):
        pltpu.make_async_copy(dst_hbm.at[0], dsts[q], sem_d[q]).wait()

    def fire_g(q, b):
        pltpu.async_copy(h_hbm.at[keys[q]], rin[b], sem_g[b])

    def wait_g(b):
        pltpu.make_async_copy(h_hbm.at[keys[0]], rin[b], sem_g[b]).wait()

    def fire_s(q, b):
        pltpu.async_copy(rout[b], agg_sh.at[dsts[q]], sem_s[b], add=True)

    def wait_s(b):
        pltpu.make_async_copy(rout[b], agg_sh.at[dsts[0]], sem_s[b]).wait()

    for q in range(6):
        fire_ki(q, q)
    for b in range(3):
        fire_d(b, b)
    for b in range(3):
        wait_ki(b)
        fire_g(b, b)

    def zb(i, _):
        r = i // (_HD // 16)
        g = i % (_HD // 16)
        zbuf[r, pl.ds(g * 16, 16)] = jnp.zeros((16,), jnp.float32)
        return 0
    lax.fori_loop(0, zbuf.shape[0] * (_HD // 16), zb, 0)

    def zfire(j, _):
        pltpu.async_copy(
            zbuf, agg_sh.at[pl.ds(s * _ROWS_PER_TILE + j * zbuf.shape[0],
                                  zbuf.shape[0])], sem_z)
        return 0
    lax.fori_loop(0, _ROWS_PER_TILE // zbuf.shape[0], zfire, 0)

    def zdrain(j, _):
        pltpu.make_async_copy(zbuf, agg_sh.at[pl.ds(0, zbuf.shape[0])],
                              sem_z).wait()
        return 0
    lax.fori_loop(0, _ROWS_PER_TILE // zbuf.shape[0], zdrain, 0)
    plsc.subcore_barrier()

    def superstep(it, _):
        for j in range(6):
            cc = it * 6 + j
            b = j % 3
            wait_g(b)

            @pl.when(cc >= 3)
            def _():
                wait_s(b)

            z = jnp.where(cc < _NCHUNK, 1.0, 0.0)

            def scale(kk, _):
                inv16 = invs[j][pl.ds(kk * 16, 16)] * z
                for jj in range(16):
                    f = inv16[jj]
                    row = kk * 16 + jj
                    for g in range(_HD // 32):
                        w16 = rin[b][row, pl.ds(g * 16, 16)]
                        a0 = lax.bitcast_convert_type(w16 << 16, jnp.float32)
                        a1 = lax.bitcast_convert_type(
                            w16 & jnp.int32(-65536), jnp.float32)
                        rout[b][row, pl.ds(g * 32, 16)] = a0 * f
                        rout[b][row, pl.ds(g * 32 + 16, 16)] = a1 * f
                return 0
            lax.fori_loop(0, _K // 16, scale, 0)

            @pl.when(cc + 3 < _NTOT)
            def _():
                wait_ki((j + 3) % 6)
                fire_g((j + 3) % 6, b)

            wait_d(j)
            fire_s(j, b)

            @pl.when(cc + 6 < _NTOT)
            def _():
                fire_ki(cc + 6, j)

            @pl.when(cc + 3 < _NTOT)
            def _():
                fire_d(cc + 3, (j + 3) % 6)
        return 0
    lax.fori_loop(0, _NTOT // 6, superstep, 0)

    for b in range(3):
        wait_s(b)
    plsc.subcore_barrier()

    r0 = s * _ROWS_PER_TILE
    pltpu.sync_copy(agg_sh.at[pl.ds(r0, _ROWS_PER_TILE)],
                    out_hbm.at[pl.ds(r0, _ROWS_PER_TILE),
                               pl.ds(c * _HD, _HD)])


def _edge_agg(h_flat, key2, dst2, inv2):
    mesh = plsc.VectorSubcoreMesh(core_axis_name="c", subcore_axis_name="s")
    f = pl.kernel(
        _agg_body,
        out_type=jax.ShapeDtypeStruct((_NP, _DIM), jnp.float32),
        mesh=mesh,
        scratch_types=[
            pltpu.VMEM_SHARED((_NP, _HD), jnp.float32),
            [pltpu.VMEM((_K, _HD // 2), jnp.int32) for _ in range(3)],
            [pltpu.VMEM((_K, _HD), jnp.float32) for _ in range(3)],
            [pltpu.VMEM((_K,), jnp.int32) for _ in range(6)],
            [pltpu.VMEM((_K,), jnp.float32) for _ in range(6)],
            [pltpu.VMEM((_K,), jnp.int32) for _ in range(6)],
            pltpu.VMEM((8, _HD), jnp.float32),
            pltpu.SemaphoreType.DMA,
            [pltpu.SemaphoreType.DMA for _ in range(3)],
            [pltpu.SemaphoreType.DMA for _ in range(3)],
            [pltpu.SemaphoreType.DMA for _ in range(6)],
            [pltpu.SemaphoreType.DMA for _ in range(6)],
        ],
        compiler_params=pltpu.CompilerParams(use_tc_tiling_on_sc=False),
    )
    return f(h_flat, key2, dst2, inv2)


def kernel(entity, edge_index, edge_type, entity_table, basis1, comp1, root1,
           bias1, basis2, comp2, root2, bias2):
    key2r, ckeyr, dstr = _prep_edges(edge_index.astype(jnp.int32),
                                     edge_type.astype(jnp.int32))
    key2 = key2r.reshape(_NC, _ECHUNK, _K)
    dst2 = dstr.reshape(_ECHUNK, _K)
    ckey2 = ckeyr.reshape(_ECHUNK, _K)

    x0 = jnp.pad(entity_table, ((0, _NP - _N_ENT), (0, 0)))

    perm = jnp.asarray(_PERM)
    w1 = _wstack(comp1, basis1, perm)
    w2 = _wstack(comp2, basis2, perm)
    wstack = jnp.concatenate([w1, root1[None], w2, root2[None]], axis=0)

    inv2 = _edge_norms(ckey2)

    nrows = _N_REL * _NP * 2

    def as_table(hb):
        pairs = hb.reshape(_N_REL, _NP, _HD, 2)
        return lax.bitcast_convert_type(pairs, jnp.int32).reshape(nrows,
                                                                  _HD // 2)

    hb1, rt1 = _rel_matmul(x0, wstack, 0)
    p1 = _edge_agg(as_table(hb1), key2, dst2, inv2)
    hb2, rt2 = _rel_matmul_fused(p1, rt1, bias1, wstack, _N_REL + 1)
    p2 = _edge_agg(as_table(hb2), key2, dst2, inv2)
    out = _combine(p2, rt2, bias2)

    return out[:_N_ENT]


# speedup vs baseline: 3.3533x; 3.3533x over previous
import functools

import jax
import jax.numpy as jnp
from jax import lax
from jax.experimental import pallas as pl
from jax.experimental.pallas import tpu as pltpu
from jax.experimental.pallas import tpu_sc as plsc

_N_ENT = 10000
_N_REL = 16
_DIM = 128
_HD = _DIM // 2
_N_EDGES = 320000

_NC, _NS = 2, 16
_NP = 10112
_NB = _NP // 128
_NKEY = _N_ENT * _N_REL
_NT = (_N_REL + 1) * _NP

_K = 80
_EPT = _N_EDGES // _NS
_NCHUNK = _EPT // _K
_NRING = 5
_ECHUNK = _N_EDGES // _K

_ROWS_PER_TILE = _NP // _NS


def _wstack_body(comp_ref, basis_ref, o_ref):
    o_ref[...] = jnp.dot(comp_ref[...], basis_ref[...],
                         preferred_element_type=jnp.float32)


def _wstack(comp, basis):
    return pl.pallas_call(
        _wstack_body,
        out_shape=jax.ShapeDtypeStruct((_N_REL, _DIM * _DIM), jnp.float32),
    )(comp, basis.reshape(4, _DIM * _DIM))


_BR = _NP // 8


def _prep_body(ei_ref, et_ref, okey_ref, ock_ref, odst_ref):
    src = ei_ref[0]
    dst = ei_ref[1]
    rel = et_ref[...]
    key = (rel * _NP + src) * 2
    okey_ref[0] = key
    okey_ref[1] = key + 1
    ock_ref[...] = dst * _N_REL + rel
    odst_ref[...] = dst


def _prep_edges(edge_index, edge_type):
    er = _N_EDGES // 128
    return pl.pallas_call(
        _prep_body,
        out_shape=[
            jax.ShapeDtypeStruct((_NC, er, 128), jnp.int32),
            jax.ShapeDtypeStruct((er, 128), jnp.int32),
            jax.ShapeDtypeStruct((er, 128), jnp.int32),
        ],
    )(edge_index.reshape(2, er, 128), edge_type.reshape(er, 128))


def _relmm_body(x_ref, w_ref, o_ref, *, off):
    x = x_ref[...]
    for r in range(_N_REL + 1):
        o_ref[r] = jnp.dot(x, w_ref[off + r],
                           preferred_element_type=jnp.float32)


def _rel_matmul(x_pad, wstack, off):
    return pl.pallas_call(
        functools.partial(_relmm_body, off=off),
        grid=(8,),
        in_specs=[
            pl.BlockSpec((_BR, _DIM), lambda i: (i, 0)),
            pl.BlockSpec((2 * _N_REL + 2, _DIM, _DIM), lambda i: (0, 0, 0)),
        ],
        out_specs=pl.BlockSpec((_N_REL + 1, _BR, _DIM), lambda i: (0, i, 0)),
        out_shape=jax.ShapeDtypeStruct((_N_REL + 1, _NP, _DIM), jnp.float32),
    )(x_pad, wstack)


def _relmm_fused_body(p_ref, h_ref, b_ref, w_ref, o_ref, *, off):
    x = jnp.maximum(p_ref[...] + h_ref[0] + b_ref[...], 0.0)
    for r in range(_N_REL + 1):
        o_ref[r] = jnp.dot(x, w_ref[off + r],
                           preferred_element_type=jnp.float32)


def _rel_matmul_fused(p_flat, h_prev, bias, wstack, off):
    return pl.pallas_call(
        functools.partial(_relmm_fused_body, off=off),
        grid=(8,),
        in_specs=[
            pl.BlockSpec((_BR, _DIM), lambda i: (i, 0)),
            pl.BlockSpec((1, _BR, _DIM), lambda i: (_N_REL, i, 0)),
            pl.BlockSpec((1, _DIM), lambda i: (0, 0)),
            pl.BlockSpec((2 * _N_REL + 2, _DIM, _DIM), lambda i: (0, 0, 0)),
        ],
        out_specs=pl.BlockSpec((_N_REL + 1, _BR, _DIM), lambda i: (0, i, 0)),
        out_shape=jax.ShapeDtypeStruct((_N_REL + 1, _NP, _DIM), jnp.float32),
    )(p_flat, h_prev, bias.reshape(1, _DIM), wstack)


def _combine_body(p_ref, h_ref, b_ref, o_ref):
    o_ref[...] = p_ref[...] + h_ref[0] + b_ref[...]


def _combine(p_flat, h, bias):
    return pl.pallas_call(
        _combine_body,
        grid=(8,),
        in_specs=[
            pl.BlockSpec((_BR, _DIM), lambda i: (i, 0)),
            pl.BlockSpec((1, _BR, _DIM), lambda i: (_N_REL, i, 0)),
            pl.BlockSpec((1, _DIM), lambda i: (0, 0)),
        ],
        out_specs=pl.BlockSpec((_BR, _DIM), lambda i: (i, 0)),
        out_shape=jax.ShapeDtypeStruct((_NP, _DIM), jnp.float32),
    )(p_flat, h, bias.reshape(1, _DIM))


def _norm_body(ckey_hbm, inv_hbm, counts_sh, zbuf, ckeys_v, cnts_v, ones_v,
               sem_s, sem_g):
    c = lax.axis_index("c")
    s = lax.axis_index("s")

    def zb(i, _):
        zbuf[pl.ds(i * 16, 16)] = jnp.zeros((16,), jnp.float32)
        return 0
    lax.fori_loop(0, zbuf.shape[0] // 16, zb, 0)

    def ob(i, _):
        ones_v[pl.ds(i * 16, 16)] = jnp.ones((16,), jnp.float32)
        return 0
    lax.fori_loop(0, _K // 16, ob, 0)

    pltpu.sync_copy(zbuf, counts_sh.at[pl.ds(s * (_NKEY // _NS),
                                             _NKEY // _NS)])
    plsc.subcore_barrier()

    pltpu.sync_copy(ckey_hbm.at[pl.ds(s * _NCHUNK, _NCHUNK)], ckeys_v)

    def f1(cc, _):
        pltpu.async_copy(ones_v, counts_sh.at[ckeys_v.at[cc]], sem_s,
                         add=True)
        return 0
    lax.fori_loop(0, _NCHUNK, f1, 0)

    def d1(cc, _):
        pltpu.make_async_copy(ones_v, counts_sh.at[ckeys_v.at[0]],
                              sem_s).wait()
        return 0
    lax.fori_loop(0, _NCHUNK, d1, 0)
    plsc.subcore_barrier()

    nch2 = _NCHUNK // 2
    row0 = (c * _NS + s) * nch2
    pltpu.sync_copy(ckey_hbm.at[pl.ds(row0, nch2)],
                    ckeys_v.at[pl.ds(0, nch2)])

    def f2(cc, _):
        pltpu.async_copy(counts_sh.at[ckeys_v.at[cc]], cnts_v.at[cc], sem_g)
        return 0
    lax.fori_loop(0, nch2, f2, 0)

    def d2(cc, _):
        pltpu.make_async_copy(counts_sh.at[ckeys_v.at[0]], cnts_v.at[0],
                              sem_g).wait()
        return 0
    lax.fori_loop(0, nch2, d2, 0)

    def inv(i, _):
        for g in range(_K // 16):
            sl = pl.ds(g * 16, 16)
            cnts_v[i, sl] = 1.0 / jnp.maximum(cnts_v[i, sl], 1.0)
        return 0
    lax.fori_loop(0, nch2, inv, 0)

    pltpu.sync_copy(cnts_v, inv_hbm.at[pl.ds(row0, nch2)])


def _edge_norms(ckey2):
    mesh = plsc.VectorSubcoreMesh(core_axis_name="c", subcore_axis_name="s")
    f = pl.kernel(
        _norm_body,
        out_type=jax.ShapeDtypeStruct((_ECHUNK, _K), jnp.float32),
        mesh=mesh,
        scratch_types=[
            pltpu.VMEM_SHARED((_NKEY,), jnp.float32),
            pltpu.VMEM((_NKEY // _NS,), jnp.float32),
            pltpu.VMEM((_NCHUNK, _K), jnp.int32),
            pltpu.VMEM((_NCHUNK // 2, _K), jnp.float32),
            pltpu.VMEM((_K,), jnp.float32),
            pltpu.SemaphoreType.DMA,
            pltpu.SemaphoreType.DMA,
        ],
        compiler_params=pltpu.CompilerParams(use_tc_tiling_on_sc=False),
    )
    return f(ckey2)


_NTOT = 252


def _agg_body(h_hbm, key_hbm, dst_hbm, inv_hbm, out_hbm, agg_sh,
              rin, rout, keys, invs, dsts, zbuf,
              sem_z, sem_g, sem_s, sem_i, sem_d):
    c = lax.axis_index("c")
    s = lax.axis_index("s")
    row_base = s * _NCHUNK

    def crow(cc):
        return row_base + jnp.minimum(cc, _NCHUNK - 1)

    def fire_ki(cc, q):
        r = crow(cc)
        pltpu.async_copy(key_hbm.at[c, r], keys[q], sem_i[q])
        pltpu.async_copy(inv_hbm.at[r], invs[q], sem_i[q])

    def wait_ki(q):
        pltpu.make_async_copy(key_hbm.at[c, 0], keys[q], sem_i[q]).wait()
        pltpu.make_async_copy(inv_hbm.at[0], invs[q], sem_i[q]).wait()

    def fire_d(cc, q):
        pltpu.async_copy(dst_hbm.at[crow(cc)], dsts[q], sem_d[q])

    def wait_d(q):
        pltpu.make_async_copy(dst_hbm.at[0], dsts[q], sem_d[q]).wait()

    def fire_g(q, b):
        pltpu.async_copy(h_hbm.at[keys[q]], rin[b], sem_g[b])

    def wait_g(b):
        pltpu.make_async_copy(h_hbm.at[keys[0]], rin[b], sem_g[b]).wait()

    def fire_s(q, b):
        pltpu.async_copy(rout[b], agg_sh.at[dsts[q]], sem_s[b], add=True)

    def wait_s(b):
        pltpu.make_async_copy(rout[b], agg_sh.at[dsts[0]], sem_s[b]).wait()

    for q in range(6):
        fire_ki(q, q)
    for b in range(3):
        fire_d(b, b)
    for b in range(3):
        wait_ki(b)
        fire_g(b, b)

    def zb(i, _):
        r = i // (_HD // 16)
        g = i % (_HD // 16)
        zbuf[r, pl.ds(g * 16, 16)] = jnp.zeros((16,), jnp.float32)
        return 0
    lax.fori_loop(0, zbuf.shape[0] * (_HD // 16), zb, 0)

    def zfire(j, _):
        pltpu.async_copy(
            zbuf, agg_sh.at[pl.ds(s * _ROWS_PER_TILE + j * zbuf.shape[0],
                                  zbuf.shape[0])], sem_z)
        return 0
    lax.fori_loop(0, _ROWS_PER_TILE // zbuf.shape[0], zfire, 0)

    def zdrain(j, _):
        pltpu.make_async_copy(zbuf, agg_sh.at[pl.ds(0, zbuf.shape[0])],
                              sem_z).wait()
        return 0
    lax.fori_loop(0, _ROWS_PER_TILE // zbuf.shape[0], zdrain, 0)
    plsc.subcore_barrier()

    def superstep(it, _):
        for j in range(6):
            cc = it * 6 + j
            b = j % 3
            wait_g(b)

            @pl.when(cc >= 3)
            def _():
                wait_s(b)

            z = jnp.where(cc < _NCHUNK, 1.0, 0.0)

            def scale(kk, _):
                inv16 = invs[j][pl.ds(kk * 16, 16)] * z
                for jj in range(16):
                    f = inv16[jj]
                    row = kk * 16 + jj
                    for g in range(_HD // 16):
                        sl = pl.ds(g * 16, 16)
                        rout[b][row, sl] = rin[b][row, sl] * f
                return 0
            lax.fori_loop(0, _K // 16, scale, 0)

            @pl.when(cc + 3 < _NTOT)
            def _():
                wait_ki((j + 3) % 6)
                fire_g((j + 3) % 6, b)

            wait_d(j)
            fire_s(j, b)

            @pl.when(cc + 6 < _NTOT)
            def _():
                fire_ki(cc + 6, j)

            @pl.when(cc + 3 < _NTOT)
            def _():
                fire_d(cc + 3, (j + 3) % 6)
        return 0
    lax.fori_loop(0, _NTOT // 6, superstep, 0)

    for b in range(3):
        wait_s(b)
    plsc.subcore_barrier()

    r0 = s * _ROWS_PER_TILE
    pltpu.sync_copy(agg_sh.at[pl.ds(r0, _ROWS_PER_TILE)],
                    out_hbm.at[pl.ds(r0, _ROWS_PER_TILE),
                               pl.ds(c * _HD, _HD)])


def _edge_agg(h_flat, key2, dst2, inv2):
    mesh = plsc.VectorSubcoreMesh(core_axis_name="c", subcore_axis_name="s")
    f = pl.kernel(
        _agg_body,
        out_type=jax.ShapeDtypeStruct((_NP, _DIM), jnp.float32),
        mesh=mesh,
        scratch_types=[
            pltpu.VMEM_SHARED((_NP, _HD), jnp.float32),
            [pltpu.VMEM((_K, _HD), jnp.float32) for _ in range(3)],
            [pltpu.VMEM((_K, _HD), jnp.float32) for _ in range(3)],
            [pltpu.VMEM((_K,), jnp.int32) for _ in range(6)],
            [pltpu.VMEM((_K,), jnp.float32) for _ in range(6)],
            [pltpu.VMEM((_K,), jnp.int32) for _ in range(6)],
            pltpu.VMEM((8, _HD), jnp.float32),
            pltpu.SemaphoreType.DMA,
            [pltpu.SemaphoreType.DMA for _ in range(3)],
            [pltpu.SemaphoreType.DMA for _ in range(3)],
            [pltpu.SemaphoreType.DMA for _ in range(6)],
            [pltpu.SemaphoreType.DMA for _ in range(6)],
        ],
        compiler_params=pltpu.CompilerParams(use_tc_tiling_on_sc=False),
    )
    return f(h_flat, key2, dst2, inv2)


def kernel(entity, edge_index, edge_type, entity_table, basis1, comp1, root1,
           bias1, basis2, comp2, root2, bias2):
    key2r, ckeyr, dstr = _prep_edges(edge_index.astype(jnp.int32),
                                     edge_type.astype(jnp.int32))
    key2 = key2r.reshape(_NC, _ECHUNK, _K)
    dst2 = dstr.reshape(_ECHUNK, _K)
    ckey2 = ckeyr.reshape(_ECHUNK, _K)

    x0 = jnp.pad(entity_table, ((0, _NP - _N_ENT), (0, 0)))

    w1 = _wstack(comp1, basis1).reshape(_N_REL, _DIM, _DIM)
    w2 = _wstack(comp2, basis2).reshape(_N_REL, _DIM, _DIM)
    wstack = jnp.concatenate([w1, root1[None], w2, root2[None]], axis=0)

    inv2 = _edge_norms(ckey2)

    h1 = _rel_matmul(x0, wstack, 0)
    p1 = _edge_agg(h1.reshape(_NT * 2, _HD), key2, dst2, inv2)
    h2 = _rel_matmul_fused(p1, h1, bias1, wstack, _N_REL + 1)
    p2 = _edge_agg(h2.reshape(_NT * 2, _HD), key2, dst2, inv2)
    out = _combine(p2, h2, bias2)

    return out[:_N_ENT]


# speedup vs baseline: 3.3875x; 1.0102x over previous
import functools

import jax
import jax.numpy as jnp
from jax import lax
from jax.experimental import pallas as pl
from jax.experimental.pallas import tpu as pltpu
from jax.experimental.pallas import tpu_sc as plsc

_N_ENT = 10000
_N_REL = 16
_DIM = 128
_HD = _DIM // 2
_N_EDGES = 320000

_NC, _NS = 2, 16
_NP = 10112
_NB = _NP // 128
_NKEY = _N_ENT * _N_REL
_NT = (_N_REL + 1) * _NP

_K = 80
_EPT = _N_EDGES // _NS
_NCHUNK = _EPT // _K
_NRING = 5
_ECHUNK = _N_EDGES // _K

_ROWS_PER_TILE = _NP // _NS


def _wstack_body(comp_ref, basis_ref, o_ref):
    o_ref[...] = jnp.dot(comp_ref[...], basis_ref[...],
                         preferred_element_type=jnp.float32)


def _wstack(comp, basis):
    return pl.pallas_call(
        _wstack_body,
        out_shape=jax.ShapeDtypeStruct((_N_REL, _DIM * _DIM), jnp.float32),
    )(comp, basis.reshape(4, _DIM * _DIM))


_BR = _NP // 8


def _prep_body(ei_ref, et_ref, okey_ref, ock_ref, odst_ref):
    src = ei_ref[0]
    dst = ei_ref[1]
    rel = et_ref[...]
    key = (rel * _NP + src) * 2
    okey_ref[0] = key
    okey_ref[1] = key + 1
    ock_ref[...] = dst * _N_REL + rel
    odst_ref[...] = dst


def _prep_edges(edge_index, edge_type):
    er = _N_EDGES // 128
    return pl.pallas_call(
        _prep_body,
        out_shape=[
            jax.ShapeDtypeStruct((_NC, er, 128), jnp.int32),
            jax.ShapeDtypeStruct((er, 128), jnp.int32),
            jax.ShapeDtypeStruct((er, 128), jnp.int32),
        ],
    )(edge_index.reshape(2, er, 128), edge_type.reshape(er, 128))


def _relmm_body(x_ref, w_ref, o_ref, *, off):
    x = x_ref[...]
    for r in range(_N_REL + 1):
        o_ref[r] = jnp.dot(x, w_ref[off + r],
                           preferred_element_type=jnp.float32)


def _rel_matmul(x_pad, wstack, off):
    return pl.pallas_call(
        functools.partial(_relmm_body, off=off),
        grid=(8,),
        in_specs=[
            pl.BlockSpec((_BR, _DIM), lambda i: (i, 0)),
            pl.BlockSpec((2 * _N_REL + 2, _DIM, _DIM), lambda i: (0, 0, 0)),
        ],
        out_specs=pl.BlockSpec((_N_REL + 1, _BR, _DIM), lambda i: (0, i, 0)),
        out_shape=jax.ShapeDtypeStruct((_N_REL + 1, _NP, _DIM), jnp.float32),
    )(x_pad, wstack)


def _relmm_fused_body(p_ref, h_ref, b_ref, w_ref, o_ref, *, off):
    x = jnp.maximum(p_ref[...] + h_ref[0] + b_ref[...], 0.0)
    for r in range(_N_REL + 1):
        o_ref[r] = jnp.dot(x, w_ref[off + r],
                           preferred_element_type=jnp.float32)


def _rel_matmul_fused(p_flat, h_prev, bias, wstack, off):
    return pl.pallas_call(
        functools.partial(_relmm_fused_body, off=off),
        grid=(8,),
        in_specs=[
            pl.BlockSpec((_BR, _DIM), lambda i: (i, 0)),
            pl.BlockSpec((1, _BR, _DIM), lambda i: (_N_REL, i, 0)),
            pl.BlockSpec((1, _DIM), lambda i: (0, 0)),
            pl.BlockSpec((2 * _N_REL + 2, _DIM, _DIM), lambda i: (0, 0, 0)),
        ],
        out_specs=pl.BlockSpec((_N_REL + 1, _BR, _DIM), lambda i: (0, i, 0)),
        out_shape=jax.ShapeDtypeStruct((_N_REL + 1, _NP, _DIM), jnp.float32),
    )(p_flat, h_prev, bias.reshape(1, _DIM), wstack)


def _combine_body(p_ref, h_ref, b_ref, o_ref):
    o_ref[...] = p_ref[...] + h_ref[0] + b_ref[...]


def _combine(p_flat, h, bias):
    return pl.pallas_call(
        _combine_body,
        grid=(10,),
        in_specs=[
            pl.BlockSpec((_N_ENT // 10, _DIM), lambda i: (i, 0)),
            pl.BlockSpec((1, _N_ENT // 10, _DIM), lambda i: (_N_REL, i, 0)),
            pl.BlockSpec((1, _DIM), lambda i: (0, 0)),
        ],
        out_specs=pl.BlockSpec((_N_ENT // 10, _DIM), lambda i: (i, 0)),
        out_shape=jax.ShapeDtypeStruct((_N_ENT, _DIM), jnp.float32),
    )(p_flat, h, bias.reshape(1, _DIM))


def _norm_body(ckey_hbm, inv_hbm, counts_sh, zbuf, ckeys_v, cnts_v, ones_v,
               sem_s, sem_g):
    c = lax.axis_index("c")
    s = lax.axis_index("s")

    def zb(i, _):
        zbuf[pl.ds(i * 16, 16)] = jnp.zeros((16,), jnp.float32)
        return 0
    lax.fori_loop(0, zbuf.shape[0] // 16, zb, 0)

    def ob(i, _):
        ones_v[pl.ds(i * 16, 16)] = jnp.ones((16,), jnp.float32)
        return 0
    lax.fori_loop(0, _K // 16, ob, 0)

    pltpu.sync_copy(zbuf, counts_sh.at[pl.ds(s * (_NKEY // _NS),
                                             _NKEY // _NS)])
    plsc.subcore_barrier()

    pltpu.sync_copy(ckey_hbm.at[pl.ds(s * _NCHUNK, _NCHUNK)], ckeys_v)

    def f1(cc, _):
        pltpu.async_copy(ones_v, counts_sh.at[ckeys_v.at[cc]], sem_s,
                         add=True)
        return 0
    lax.fori_loop(0, _NCHUNK, f1, 0)

    def d1(cc, _):
        pltpu.make_async_copy(ones_v, counts_sh.at[ckeys_v.at[0]],
                              sem_s).wait()
        return 0
    lax.fori_loop(0, _NCHUNK, d1, 0)
    plsc.subcore_barrier()

    nch2 = _NCHUNK // 2
    row0 = (c * _NS + s) * nch2
    pltpu.sync_copy(ckey_hbm.at[pl.ds(row0, nch2)],
                    ckeys_v.at[pl.ds(0, nch2)])

    def f2(cc, _):
        pltpu.async_copy(counts_sh.at[ckeys_v.at[cc]], cnts_v.at[cc], sem_g)
        return 0
    lax.fori_loop(0, nch2, f2, 0)

    def d2(cc, _):
        pltpu.make_async_copy(counts_sh.at[ckeys_v.at[0]], cnts_v.at[0],
                              sem_g).wait()
        return 0
    lax.fori_loop(0, nch2, d2, 0)

    def inv(i, _):
        for g in range(_K // 16):
            sl = pl.ds(g * 16, 16)
            cnts_v[i, sl] = 1.0 / jnp.maximum(cnts_v[i, sl], 1.0)
        return 0
    lax.fori_loop(0, nch2, inv, 0)

    pltpu.sync_copy(cnts_v, inv_hbm.at[pl.ds(row0, nch2)])


def _edge_norms(ckey2):
    mesh = plsc.VectorSubcoreMesh(core_axis_name="c", subcore_axis_name="s")
    f = pl.kernel(
        _norm_body,
        out_type=jax.ShapeDtypeStruct((_ECHUNK, _K), jnp.float32),
        mesh=mesh,
        scratch_types=[
            pltpu.VMEM_SHARED((_NKEY,), jnp.float32),
            pltpu.VMEM((_NKEY // _NS,), jnp.float32),
            pltpu.VMEM((_NCHUNK, _K), jnp.int32),
            pltpu.VMEM((_NCHUNK // 2, _K), jnp.float32),
            pltpu.VMEM((_K,), jnp.float32),
            pltpu.SemaphoreType.DMA,
            pltpu.SemaphoreType.DMA,
        ],
        compiler_params=pltpu.CompilerParams(use_tc_tiling_on_sc=False),
    )
    return f(ckey2)


_NTOT = 252


def _agg_body(h_hbm, key_hbm, dst_hbm, inv_hbm, out_hbm, agg_sh,
              rin, rout, keys, invs, dsts, zbuf,
              sem_z, sem_g, sem_s, sem_i, sem_d):
    c = lax.axis_index("c")
    s = lax.axis_index("s")
    row_base = s * _NCHUNK

    def crow(cc):
        return row_base + jnp.minimum(cc, _NCHUNK - 1)

    def fire_ki(cc, q):
        r = crow(cc)
        pltpu.async_copy(key_hbm.at[c, r], keys[q], sem_i[q])
        pltpu.async_copy(inv_hbm.at[r], invs[q], sem_i[q])

    def wait_ki(q):
        pltpu.make_async_copy(key_hbm.at[c, 0], keys[q], sem_i[q]).wait()
        pltpu.make_async_copy(inv_hbm.at[0], invs[q], sem_i[q]).wait()

    def fire_d(cc, q):
        pltpu.async_copy(dst_hbm.at[crow(cc)], dsts[q], sem_d[q])

    def wait_d(q):
        pltpu.make_async_copy(dst_hbm.at[0], dsts[q], sem_d[q]).wait()

    def fire_g(q, b):
        pltpu.async_copy(h_hbm.at[keys[q]], rin[b], sem_g[b])

    def wait_g(b):
        pltpu.make_async_copy(h_hbm.at[keys[0]], rin[b], sem_g[b]).wait()

    def fire_s(q, b):
        pltpu.async_copy(rout[b], agg_sh.at[dsts[q]], sem_s[b], add=True)

    def wait_s(b):
        pltpu.make_async_copy(rout[b], agg_sh.at[dsts[0]], sem_s[b]).wait()

    for q in range(6):
        fire_ki(q, q)
    for b in range(3):
        fire_d(b, b)
    for b in range(3):
        wait_ki(b)
        fire_g(b, b)

    def zb(i, _):
        r = i // (_HD // 16)
        g = i % (_HD // 16)
        zbuf[r, pl.ds(g * 16, 16)] = jnp.zeros((16,), jnp.float32)
        return 0
    lax.fori_loop(0, zbuf.shape[0] * (_HD // 16), zb, 0)

    def zfire(j, _):
        pltpu.async_copy(
            zbuf, agg_sh.at[pl.ds(s * _ROWS_PER_TILE + j * zbuf.shape[0],
                                  zbuf.shape[0])], sem_z)
        return 0
    lax.fori_loop(0, _ROWS_PER_TILE // zbuf.shape[0], zfire, 0)

    def zdrain(j, _):
        pltpu.make_async_copy(zbuf, agg_sh.at[pl.ds(0, zbuf.shape[0])],
                              sem_z).wait()
        return 0
    lax.fori_loop(0, _ROWS_PER_TILE // zbuf.shape[0], zdrain, 0)
    plsc.subcore_barrier()

    def superstep(it, _):
        for j in range(6):
            cc = it * 6 + j
            b = j % 3
            wait_g(b)

            @pl.when(cc >= 3)
            def _():
                wait_s(b)

            z = jnp.where(cc < _NCHUNK, 1.0, 0.0)

            def scale(kk, _):
                inv16 = invs[j][pl.ds(kk * 16, 16)] * z
                for jj in range(16):
                    f = inv16[jj]
                    row = kk * 16 + jj
                    for g in range(_HD // 16):
                        sl = pl.ds(g * 16, 16)
                        rout[b][row, sl] = rin[b][row, sl] * f
                return 0
            lax.fori_loop(0, _K // 16, scale, 0)

            @pl.when(cc + 3 < _NTOT)
            def _():
                wait_ki((j + 3) % 6)
                fire_g((j + 3) % 6, b)

            wait_d(j)
            fire_s(j, b)

            @pl.when(cc + 6 < _NTOT)
            def _():
                fire_ki(cc + 6, j)

            @pl.when(cc + 3 < _NTOT)
            def _():
                fire_d(cc + 3, (j + 3) % 6)
        return 0
    lax.fori_loop(0, _NTOT // 6, superstep, 0)

    for b in range(3):
        wait_s(b)
    plsc.subcore_barrier()

    r0 = s * _ROWS_PER_TILE
    pltpu.sync_copy(agg_sh.at[pl.ds(r0, _ROWS_PER_TILE)],
                    out_hbm.at[pl.ds(r0, _ROWS_PER_TILE),
                               pl.ds(c * _HD, _HD)])


def _edge_agg(h_flat, key2, dst2, inv2):
    mesh = plsc.VectorSubcoreMesh(core_axis_name="c", subcore_axis_name="s")
    f = pl.kernel(
        _agg_body,
        out_type=jax.ShapeDtypeStruct((_NP, _DIM), jnp.float32),
        mesh=mesh,
        scratch_types=[
            pltpu.VMEM_SHARED((_NP, _HD), jnp.float32),
            [pltpu.VMEM((_K, _HD), jnp.float32) for _ in range(3)],
            [pltpu.VMEM((_K, _HD), jnp.float32) for _ in range(3)],
            [pltpu.VMEM((_K,), jnp.int32) for _ in range(6)],
            [pltpu.VMEM((_K,), jnp.float32) for _ in range(6)],
            [pltpu.VMEM((_K,), jnp.int32) for _ in range(6)],
            pltpu.VMEM((8, _HD), jnp.float32),
            pltpu.SemaphoreType.DMA,
            [pltpu.SemaphoreType.DMA for _ in range(3)],
            [pltpu.SemaphoreType.DMA for _ in range(3)],
            [pltpu.SemaphoreType.DMA for _ in range(6)],
            [pltpu.SemaphoreType.DMA for _ in range(6)],
        ],
        compiler_params=pltpu.CompilerParams(use_tc_tiling_on_sc=False),
    )
    return f(h_flat, key2, dst2, inv2)


def kernel(entity, edge_index, edge_type, entity_table, basis1, comp1, root1,
           bias1, basis2, comp2, root2, bias2):
    key2r, ckeyr, dstr = _prep_edges(edge_index.astype(jnp.int32),
                                     edge_type.astype(jnp.int32))
    key2 = key2r.reshape(_NC, _ECHUNK, _K)
    dst2 = dstr.reshape(_ECHUNK, _K)
    ckey2 = ckeyr.reshape(_ECHUNK, _K)

    x0 = jnp.pad(entity_table, ((0, _NP - _N_ENT), (0, 0)))

    w1 = _wstack(comp1, basis1).reshape(_N_REL, _DIM, _DIM)
    w2 = _wstack(comp2, basis2).reshape(_N_REL, _DIM, _DIM)
    wstack = jnp.concatenate([w1, root1[None], w2, root2[None]], axis=0)

    inv2 = _edge_norms(ckey2)

    h1 = _rel_matmul(x0, wstack, 0)
    p1 = _edge_agg(h1.reshape(_NT * 2, _HD), key2, dst2, inv2)
    h2 = _rel_matmul_fused(p1, h1, bias1, wstack, _N_REL + 1)
    p2 = _edge_agg(h2.reshape(_NT * 2, _HD), key2, dst2, inv2)
    return _combine(p2, h2, bias2)
